# baseline (speedup 1.0000x reference)
import jax
import jax.numpy as jnp
from jax import lax
from jax.experimental import pallas as pl
from jax.experimental.pallas import tpu as pltpu
from jax.experimental.pallas import tpu_sc as plsc

N_NODES = 50000
N_TBL = 50048
HALF_D = 32
E = 800000
N_LAYERS = 3
NC, NS = 2, 16
NG = 4
NE = 8

T_STREAMS = 400
E_TILE = T_STREAMS * 128
E_PAD = E_TILE * NS
IDX_ROWS = E_PAD // 128
ROWS_PER_TILE = N_TBL // NS
N_GRP = T_STREAMS // NE


def _body(ego0, cols, rows, vals, out, t0, t1, t2,
          acc, ebuf, bfb, fbuf, sem_e, sem_g, sem_s, sem_f):
    c = lax.axis_index("c")
    s = lax.axis_index("s")
    half_base = c * N_TBL + s * ROWS_PER_TILE
    ebase = s * T_STREAMS
    zeros16 = jnp.zeros((16,), jnp.float32)

    def pack_chunks(src_ref, src_base, dst_tbl):
        def issue_load(q, b, nrows):
            pltpu.async_copy(
                src_ref.at[pl.ds(src_base + q * 128, nrows)],
                fbuf.at[b].at[pl.ds(0, nrows)], sem_f.at[b])

        def pack_one(q, b, nrows):
            @pl.loop(0, nrows)
            def _pk(r):
                ba = plsc.bitcast(fbuf[b, r, 0:16], jnp.int32)
                bb = plsc.bitcast(fbuf[b, r, 16:32], jnp.int32)
                ra = jnp.right_shift(
                    ba + 0x7FFF + jnp.bitwise_and(
                        jnp.right_shift(ba, 16), 1), 16)
                ra = jnp.bitwise_and(ra, 0xFFFF)
                rb = jnp.bitwise_and(
                    bb + 0x7FFF + jnp.bitwise_and(
                        jnp.right_shift(bb, 16), 1),
                    jnp.int32(-65536))
                bfb[b, r, 0:16] = jnp.bitwise_or(ra, rb)

            pltpu.sync_copy(
                bfb.at[b].at[pl.ds(0, nrows)],
                dst_tbl.at[pl.ds(half_base + q * 128, nrows)])

        issue_load(0, 0, 128)

        @pl.loop(0, 12)
        def _pc(g):
            for b in (0, 1):
                q = 2 * g + b

                @pl.when(q <= 22)
                def _nx():
                    issue_load(q + 1, 1 - b, 128)

                pltpu.make_async_copy(
                    src_ref.at[pl.ds(src_base, 128)],
                    fbuf.at[b].at[pl.ds(0, 128)], sem_f.at[b]).wait()
                pack_one(q, b, 128)

        issue_load(24, 0, 56)
        pltpu.make_async_copy(
            src_ref.at[pl.ds(src_base, 56)],
            fbuf.at[0].at[pl.ds(0, 56)], sem_f.at[0]).wait()
        pack_one(24, 0, 56)

    pack_chunks(ego0, half_base, t0)
    plsc.subcore_barrier()

    def run_layer(src_tbl, dst_tbl):
        @pl.loop(0, 128)
        def _zb(r):
            fbuf[0, r, 0:16] = zeros16
            fbuf[0, r, 16:32] = zeros16

        for q in range(24):
            pltpu.async_copy(
                fbuf.at[0],
                acc.at[pl.ds(s * ROWS_PER_TILE + q * 128, 128)],
                sem_f.at[q % 4])
        pltpu.sync_copy(
            fbuf.at[0].at[pl.ds(0, 56)],
            acc.at[pl.ds(s * ROWS_PER_TILE + 3072, 56)])
        for q in range(24):
            pltpu.make_async_copy(
                fbuf.at[0],
                acc.at[pl.ds(s * ROWS_PER_TILE, 128)],
                sem_f.at[q % 4]).wait()
        plsc.subcore_barrier()

        def stage(kk, e):
            pltpu.async_copy(cols.at[ebase + kk], ebuf.at[e].at[0],
                             sem_e.at[e])
            pltpu.async_copy(rows.at[ebase + kk], ebuf.at[e].at[1],
                             sem_e.at[e])
            pltpu.async_copy(vals.at[ebase + kk], ebuf.at[e].at[2],
                             sem_e.at[e])

        def wait_e(e):
            pltpu.make_async_copy(cols.at[pl.ds(0, 3)], ebuf.at[e],
                                  sem_e.at[e]).wait()
            for i in range(8):
                sl = pl.ds(i * 16, 16)
                ebuf[e, 0, sl] = ebuf[e, 0, sl] + c * N_TBL

        def gather(j, e):
            pltpu.async_copy(src_tbl.at[ebuf.at[e].at[0]],
                             bfb.at[j], sem_g.at[j])

        def wait_g(j):
            pltpu.make_async_copy(src_tbl.at[ebuf.at[0].at[0]],
                                  bfb.at[j], sem_g.at[j]).wait()

        def scatter(j, e):
            pltpu.async_copy(fbuf.at[j], acc.at[ebuf.at[e].at[1]],
                             sem_s.at[j], add=True)

        def wait_s(j):
            pltpu.make_async_copy(fbuf.at[j], acc.at[ebuf.at[0].at[1]],
                                  sem_s.at[j]).wait()

        def scale(j, e):
            @pl.loop(0, 8)
            def _scale(i):
                vv = plsc.bitcast(
                    ebuf[e, 2, pl.ds(i * 16, 16)], jnp.float32)
                for q in range(16):
                    eidx = i * 16 + q
                    w = bfb[j, eidx, 0:16]
                    a = plsc.bitcast(jnp.left_shift(w, 16), jnp.float32)
                    b = plsc.bitcast(
                        jnp.bitwise_and(w, jnp.int32(-65536)), jnp.float32)
                    v = vv[q]
                    fbuf[j, eidx, 0:16] = a * v
                    fbuf[j, eidx, 16:32] = b * v

        def step(kk, j, e):
            wait_g(j)
            scale(j, e)
            scatter(j, e)

            @pl.when(kk >= 2)
            def _ws():
                wait_s((j + 2) % NG)

            @pl.when(kk + 5 <= T_STREAMS - 1)
            def _st():
                stage(kk + 5, (e + 5) % NE)

            @pl.when(kk + 3 <= T_STREAMS - 1)
            def _ga():
                wait_e((e + 3) % NE)
                gather((j + 3) % NG, (e + 3) % NE)

        for x in range(5):
            stage(x, x)
        for x in range(3):
            wait_e(x)
            gather(x, x)

        @pl.loop(0, N_GRP)
        def _grp(g):
            for m in range(NE):
                step(NE * g + m, m % NG, m)

        for j in (2, 3):
            wait_s(j)

        plsc.subcore_barrier()
        if dst_tbl is not None:
            pack_chunks(acc, s * ROWS_PER_TILE, dst_tbl)
            plsc.subcore_barrier()

    srcs = [t0, t1, t2]
    dsts = [t1, t2, None]
    for l in range(N_LAYERS):
        run_layer(srcs[l], dsts[l])

    def fin_chunk(q, nrows):
        base = s * ROWS_PER_TILE + q * 128
        tbase = half_base + q * 128
        pltpu.async_copy(ego0.at[pl.ds(tbase, nrows)],
                         fbuf.at[0].at[pl.ds(0, nrows)], sem_f.at[0])
        pltpu.async_copy(acc.at[pl.ds(base, nrows)],
                         fbuf.at[1].at[pl.ds(0, nrows)], sem_f.at[1])
        pltpu.async_copy(t1.at[pl.ds(tbase, nrows)],
                         bfb.at[0].at[pl.ds(0, nrows)], sem_f.at[2])
        pltpu.async_copy(t2.at[pl.ds(tbase, nrows)],
                         bfb.at[1].at[pl.ds(0, nrows)], sem_f.at[3])
        pltpu.make_async_copy(ego0.at[pl.ds(tbase, nrows)],
                              fbuf.at[0].at[pl.ds(0, nrows)],
                              sem_f.at[0]).wait()
        pltpu.make_async_copy(acc.at[pl.ds(base, nrows)],
                              fbuf.at[1].at[pl.ds(0, nrows)],
                              sem_f.at[1]).wait()
        pltpu.make_async_copy(t1.at[pl.ds(tbase, nrows)],
                              bfb.at[0].at[pl.ds(0, nrows)],
                              sem_f.at[2]).wait()
        pltpu.make_async_copy(t2.at[pl.ds(tbase, nrows)],
                              bfb.at[1].at[pl.ds(0, nrows)],
                              sem_f.at[3]).wait()

        @pl.loop(0, nrows)
        def _avg(r):
            w1 = bfb[0, r, 0:16]
            a1 = plsc.bitcast(jnp.left_shift(w1, 16), jnp.float32)
            b1 = plsc.bitcast(
                jnp.bitwise_and(w1, jnp.int32(-65536)), jnp.float32)
            w2 = bfb[1, r, 0:16]
            a2 = plsc.bitcast(jnp.left_shift(w2, 16), jnp.float32)
            b2 = plsc.bitcast(
                jnp.bitwise_and(w2, jnp.int32(-65536)), jnp.float32)
            fbuf[2, r, 0:16] = (
                (fbuf[0, r, 0:16] + fbuf[1, r, 0:16]) + (a1 + a2)) * 0.25
            fbuf[2, r, 16:32] = (
                (fbuf[0, r, 16:32] + fbuf[1, r, 16:32]) + (b1 + b2)) * 0.25

        pltpu.sync_copy(
            fbuf.at[2].at[pl.ds(0, nrows)],
            out.at[pl.ds(base, nrows), pl.ds(c * HALF_D, HALF_D)])

    @pl.loop(0, 24)
    def _fin(t):
        fin_chunk(t, 128)

    fin_chunk(24, 56)


_mesh = plsc.VectorSubcoreMesh(
    core_axis_name="c", subcore_axis_name="s", num_cores=NC, num_subcores=NS)

_tblb = jax.ShapeDtypeStruct((2 * N_TBL, HALF_D // 2), jnp.int32)
_out_t = jax.ShapeDtypeStruct((N_TBL, 2 * HALF_D), jnp.float32)

_gcl = pl.kernel(
    _body,
    out_type=(_out_t, _tblb, _tblb, _tblb),
    mesh=_mesh,
    compiler_params=pltpu.CompilerParams(
        use_tc_tiling_on_sc=False, needs_layout_passes=False),
    scratch_types=[
        pltpu.VMEM_SHARED((N_TBL, HALF_D), jnp.float32),
        pltpu.VMEM((NE, 3, 128), jnp.int32),
        pltpu.VMEM((NG, 128, HALF_D // 2), jnp.int32),
        pltpu.VMEM((NG, 128, HALF_D), jnp.float32),
        pltpu.SemaphoreType.DMA((NE,)),
        pltpu.SemaphoreType.DMA((NG,)),
        pltpu.SemaphoreType.DMA((NG,)),
        pltpu.SemaphoreType.DMA((4,)),
    ],
)


@jax.jit
def kernel(user_emb, item_emb, adj_rows, adj_cols, adj_vals):
    ego = jnp.concatenate([user_emb, item_emb], axis=0)
    zrows = jnp.zeros((N_TBL - N_NODES, HALF_D), jnp.float32)
    ego_h = jnp.concatenate(
        [ego[:, :HALF_D], zrows, ego[:, HALF_D:], zrows], axis=0)
    pad = E_PAD - E
    rows_p = jnp.concatenate(
        [adj_rows.astype(jnp.int32), jnp.zeros((pad,), jnp.int32)])
    cols_p = jnp.concatenate(
        [adj_cols.astype(jnp.int32), jnp.zeros((pad,), jnp.int32)])
    vals_p = jnp.concatenate([adj_vals, jnp.zeros((pad,), jnp.float32)])
    cols_r = cols_p.reshape(IDX_ROWS, 128)
    rows_r = rows_p.reshape(IDX_ROWS, 128)
    vals_r = lax.bitcast_convert_type(vals_p, jnp.int32).reshape(
        IDX_ROWS, 128)

    out, _, _, _ = _gcl(ego_h, cols_r, rows_r, vals_r)
    return out[: N_NODES // 2], out[N_NODES // 2: N_NODES]


# speedup vs baseline: 1.2166x; 1.2166x over previous
import jax
import jax.numpy as jnp
from jax import lax
from jax.experimental import pallas as pl
from jax.experimental.pallas import tpu as pltpu
from jax.experimental.pallas import tpu_sc as plsc

N_NODES = 50000
N_TBL = 50048
HALF_D = 32
E = 800000
N_LAYERS = 3
NC, NS = 2, 16
NBUF = 6

T_STREAMS = 396
E_TILE = T_STREAMS * 128
E_PAD = E_TILE * NS
IDX_ROWS = E_PAD // 128
ROWS_PER_TILE = N_TBL // NS
N_GRP = T_STREAMS // NBUF


def _body(ego0, cols, rows, vals, out, t1, t2,
          acc, ebuf, gbuf, sem_e, sem_g, sem_s, sem_f):
    c = lax.axis_index("c")
    s = lax.axis_index("s")
    half_base = c * N_TBL + s * ROWS_PER_TILE
    ebase = s * T_STREAMS
    zeros16 = jnp.zeros((16,), jnp.float32)

    def run_layer(src_tbl, dst_tbl):
        @pl.loop(0, 128)
        def _zb(r):
            gbuf[0, r, 0:16] = zeros16
            gbuf[0, r, 16:32] = zeros16

        for q in range(24):
            pltpu.async_copy(
                gbuf.at[0],
                acc.at[pl.ds(s * ROWS_PER_TILE + q * 128, 128)],
                sem_f.at[q % 4])
        pltpu.sync_copy(
            gbuf.at[0].at[pl.ds(0, 56)],
            acc.at[pl.ds(s * ROWS_PER_TILE + 3072, 56)])
        for q in range(24):
            pltpu.make_async_copy(
                gbuf.at[0],
                acc.at[pl.ds(s * ROWS_PER_TILE, 128)],
                sem_f.at[q % 4]).wait()
        plsc.subcore_barrier()

        def stage(kk, j):
            pltpu.async_copy(cols.at[ebase + kk], ebuf.at[j].at[0],
                             sem_e.at[j])
            pltpu.async_copy(rows.at[ebase + kk], ebuf.at[j].at[1],
                             sem_e.at[j])
            pltpu.async_copy(vals.at[ebase + kk], ebuf.at[j].at[2],
                             sem_e.at[j])

        def wait_e(j):
            pltpu.make_async_copy(cols.at[pl.ds(0, 3)], ebuf.at[j],
                                  sem_e.at[j]).wait()
            for i in range(8):
                sl = pl.ds(i * 16, 16)
                ebuf[j, 0, sl] = ebuf[j, 0, sl] + c * N_TBL

        def gather(j):
            pltpu.async_copy(src_tbl.at[ebuf.at[j].at[0]],
                             gbuf.at[j], sem_g.at[j])

        def wait_g(j):
            pltpu.make_async_copy(src_tbl.at[ebuf.at[j].at[0]],
                                  gbuf.at[j], sem_g.at[j]).wait()

        def scatter(j):
            pltpu.async_copy(gbuf.at[j], acc.at[ebuf.at[j].at[1]],
                             sem_s.at[j], add=True)

        def wait_s(j):
            pltpu.make_async_copy(gbuf.at[j], acc.at[ebuf.at[j].at[1]],
                                  sem_s.at[j]).wait()

        def scale(j):
            @pl.loop(0, 8)
            def _scale(i):
                vv = plsc.bitcast(
                    ebuf[j, 2, pl.ds(i * 16, 16)], jnp.float32)
                for q in range(16):
                    e = i * 16 + q
                    v = vv[q]
                    gbuf[j, e, 0:16] = gbuf[j, e, 0:16] * v
                    gbuf[j, e, 16:32] = gbuf[j, e, 16:32] * v

        def step(kk, j):
            wait_g(j)
            scale(j)
            scatter(j)

            @pl.when(kk >= 2)
            def _ws():
                wait_s((j + 4) % NBUF)

            @pl.when(kk + 4 <= T_STREAMS - 1)
            def _st():
                stage(kk + 4, (j + 4) % NBUF)

            @pl.when(kk + 3 <= T_STREAMS - 1)
            def _ga():
                wait_e((j + 3) % NBUF)
                gather((j + 3) % NBUF)

        for j in range(4):
            stage(j, j)
        for j in range(3):
            wait_e(j)
            gather(j)

        @pl.loop(0, N_GRP)
        def _grp(g):
            for j in range(NBUF):
                step(NBUF * g + j, j)

        for j in (4, 5):
            wait_s(j)

        plsc.subcore_barrier()
        if dst_tbl is not None:
            pltpu.sync_copy(
                acc.at[pl.ds(s * ROWS_PER_TILE, ROWS_PER_TILE)],
                dst_tbl.at[pl.ds(half_base, ROWS_PER_TILE)])

    srcs = [ego0, t1, t2]
    dsts = [t1, t2, None]
    for l in range(N_LAYERS):
        run_layer(srcs[l], dsts[l])

    def fin_chunk(q, nrows):
        base = s * ROWS_PER_TILE + q * 128
        tbase = half_base + q * 128
        for i, tbl in enumerate((ego0, t1, t2)):
            pltpu.async_copy(tbl.at[pl.ds(tbase, nrows)],
                             gbuf.at[i].at[pl.ds(0, nrows)], sem_f.at[i])
        pltpu.async_copy(acc.at[pl.ds(base, nrows)],
                         gbuf.at[3].at[pl.ds(0, nrows)], sem_f.at[3])
        for i, tbl in enumerate((ego0, t1, t2)):
            pltpu.make_async_copy(tbl.at[pl.ds(tbase, nrows)],
                                  gbuf.at[i].at[pl.ds(0, nrows)],
                                  sem_f.at[i]).wait()
        pltpu.make_async_copy(acc.at[pl.ds(base, nrows)],
                              gbuf.at[3].at[pl.ds(0, nrows)],
                              sem_f.at[3]).wait()

        @pl.loop(0, nrows)
        def _avg(r):
            for h in (0, 16):
                gbuf[4, r, h:h + 16] = (
                    (gbuf[0, r, h:h + 16] + gbuf[1, r, h:h + 16])
                    + (gbuf[2, r, h:h + 16] + gbuf[3, r, h:h + 16])) * 0.25

        pltpu.sync_copy(
            gbuf.at[4].at[pl.ds(0, nrows)],
            out.at[pl.ds(base, nrows), pl.ds(c * HALF_D, HALF_D)])

    @pl.loop(0, 24)
    def _fin(t):
        fin_chunk(t, 128)

    fin_chunk(24, 56)


_mesh = plsc.VectorSubcoreMesh(
    core_axis_name="c", subcore_axis_name="s", num_cores=NC, num_subcores=NS)

_tbl = jax.ShapeDtypeStruct((2 * N_TBL, HALF_D), jnp.float32)
_out_t = jax.ShapeDtypeStruct((N_TBL, 2 * HALF_D), jnp.float32)

_gcl = pl.kernel(
    _body,
    out_type=(_out_t, _tbl, _tbl),
    mesh=_mesh,
    compiler_params=pltpu.CompilerParams(
        use_tc_tiling_on_sc=False, needs_layout_passes=False),
    scratch_types=[
        pltpu.VMEM_SHARED((N_TBL, HALF_D), jnp.float32),
        pltpu.VMEM((NBUF, 3, 128), jnp.int32),
        pltpu.VMEM((NBUF, 128, HALF_D), jnp.float32),
        pltpu.SemaphoreType.DMA((NBUF,)),
        pltpu.SemaphoreType.DMA((NBUF,)),
        pltpu.SemaphoreType.DMA((NBUF,)),
        pltpu.SemaphoreType.DMA((4,)),
    ],
)


@jax.jit
def kernel(user_emb, item_emb, adj_rows, adj_cols, adj_vals):
    zrows = jnp.zeros((N_TBL - N_NODES, HALF_D), jnp.float32)
    ego_h = jnp.concatenate(
        [user_emb[:, :HALF_D], item_emb[:, :HALF_D], zrows,
         user_emb[:, HALF_D:], item_emb[:, HALF_D:], zrows], axis=0)
    pad = E_PAD - E
    rows_p = jnp.concatenate(
        [adj_rows.astype(jnp.int32), jnp.zeros((pad,), jnp.int32)])
    cols_p = jnp.concatenate(
        [adj_cols.astype(jnp.int32), jnp.zeros((pad,), jnp.int32)])
    vals_p = jnp.concatenate([adj_vals, jnp.zeros((pad,), jnp.float32)])
    cols_r = cols_p.reshape(IDX_ROWS, 128)
    rows_r = rows_p.reshape(IDX_ROWS, 128)
    vals_r = lax.bitcast_convert_type(vals_p, jnp.int32).reshape(
        IDX_ROWS, 128)

    out, _, _ = _gcl(ego_h, cols_r, rows_r, vals_r)
    return out[: N_NODES // 2], out[N_NODES // 2: N_NODES]


# speedup vs baseline: 1.2632x; 1.0382x over previous
import jax
import jax.numpy as jnp
from jax import lax
from jax.experimental import pallas as pl
from jax.experimental.pallas import tpu as pltpu
from jax.experimental.pallas import tpu_sc as plsc

N_NODES = 50000
N_TBL = 50048
HALF_D = 32
E = 800000
N_LAYERS = 3
NC, NS = 2, 16
NBUF = 6

T_STREAMS = 396
E_TILE = T_STREAMS * 128
E_PAD = E_TILE * NS
IDX_ROWS = E_PAD // 128
ROWS_PER_TILE = N_TBL // NS
N_GRP = T_STREAMS // NBUF


def _body(ego0, cols, rows, vals, out, t1, t2,
          acc, ebuf, gbuf, sem_e, sem_g, sem_s, sem_f):
    c = lax.axis_index("c")
    s = lax.axis_index("s")
    half_base = c * N_TBL + s * ROWS_PER_TILE
    ebase = s * T_STREAMS
    zeros16 = jnp.zeros((16,), jnp.float32)

    def run_layer(src_tbl, dst_tbl):
        @pl.loop(0, 128)
        def _zb(r):
            gbuf[0, r, 0:16] = zeros16
            gbuf[0, r, 16:32] = zeros16

        for q in range(24):
            pltpu.async_copy(
                gbuf.at[0],
                acc.at[pl.ds(s * ROWS_PER_TILE + q * 128, 128)],
                sem_f.at[q % 4])
        pltpu.sync_copy(
            gbuf.at[0].at[pl.ds(0, 56)],
            acc.at[pl.ds(s * ROWS_PER_TILE + 3072, 56)])
        for q in range(24):
            pltpu.make_async_copy(
                gbuf.at[0],
                acc.at[pl.ds(s * ROWS_PER_TILE, 128)],
                sem_f.at[q % 4]).wait()
        plsc.subcore_barrier()

        def stage(kk, j):
            pltpu.async_copy(cols.at[ebase + kk], ebuf.at[j].at[0],
                             sem_e.at[j])
            pltpu.async_copy(rows.at[ebase + kk], ebuf.at[j].at[1],
                             sem_e.at[j])
            pltpu.async_copy(vals.at[ebase + kk], ebuf.at[j].at[2],
                             sem_e.at[j])

        def wait_e(j):
            pltpu.make_async_copy(cols.at[pl.ds(0, 3)], ebuf.at[j],
                                  sem_e.at[j]).wait()
            for i in range(8):
                sl = pl.ds(i * 16, 16)
                ebuf[j, 0, sl] = ebuf[j, 0, sl] + c * N_TBL

        def gather(j):
            pltpu.async_copy(src_tbl.at[ebuf.at[j].at[0]],
                             gbuf.at[j], sem_g.at[j])

        def wait_g(j):
            pltpu.make_async_copy(src_tbl.at[ebuf.at[j].at[0]],
                                  gbuf.at[j], sem_g.at[j]).wait()

        def scatter(j):
            pltpu.async_copy(gbuf.at[j], acc.at[ebuf.at[j].at[1]],
                             sem_s.at[j], add=True)

        def wait_s(j):
            pltpu.make_async_copy(gbuf.at[j], acc.at[ebuf.at[j].at[1]],
                                  sem_s.at[j]).wait()

        def scale(j):
            @pl.loop(0, 8)
            def _scale(i):
                vv = plsc.bitcast(
                    ebuf[j, 2, pl.ds(i * 16, 16)], jnp.float32)
                for q in range(16):
                    e = i * 16 + q
                    v = vv[q]
                    gbuf[j, e, 0:16] = gbuf[j, e, 0:16] * v
                    gbuf[j, e, 16:32] = gbuf[j, e, 16:32] * v

        def step(kk, j):
            wait_g(j)
            scale(j)
            scatter(j)

            @pl.when(kk >= 2)
            def _ws():
                wait_s((j + 4) % NBUF)

            @pl.when(kk + 4 <= T_STREAMS - 1)
            def _st():
                stage(kk + 4, (j + 4) % NBUF)

            @pl.when(kk + 3 <= T_STREAMS - 1)
            def _ga():
                wait_e((j + 3) % NBUF)
                gather((j + 3) % NBUF)

        for j in range(4):
            stage(j, j)
        for j in range(3):
            wait_e(j)
            gather(j)

        @pl.loop(0, N_GRP)
        def _grp(g):
            for j in range(NBUF):
                step(NBUF * g + j, j)

        for j in (4, 5):
            wait_s(j)

        plsc.subcore_barrier()
        if dst_tbl is not None:
            pltpu.sync_copy(
                acc.at[pl.ds(s * ROWS_PER_TILE, ROWS_PER_TILE)],
                dst_tbl.at[pl.ds(half_base, ROWS_PER_TILE)])

    srcs = [ego0, t1, t2]
    dsts = [t1, t2, None]
    for l in range(N_LAYERS):
        run_layer(srcs[l], dsts[l])

    def fin_chunk(q, nrows):
        base = s * ROWS_PER_TILE + q * 128
        tbase = half_base + q * 128
        for i, tbl in enumerate((ego0, t1, t2)):
            pltpu.async_copy(tbl.at[pl.ds(tbase, nrows)],
                             gbuf.at[i].at[pl.ds(0, nrows)], sem_f.at[i])
        pltpu.async_copy(acc.at[pl.ds(base, nrows)],
                         gbuf.at[3].at[pl.ds(0, nrows)], sem_f.at[3])
        for i, tbl in enumerate((ego0, t1, t2)):
            pltpu.make_async_copy(tbl.at[pl.ds(tbase, nrows)],
                                  gbuf.at[i].at[pl.ds(0, nrows)],
                                  sem_f.at[i]).wait()
        pltpu.make_async_copy(acc.at[pl.ds(base, nrows)],
                              gbuf.at[3].at[pl.ds(0, nrows)],
                              sem_f.at[3]).wait()

        @pl.loop(0, nrows)
        def _avg(r):
            for h in (0, 16):
                gbuf[4, r, h:h + 16] = (
                    (gbuf[0, r, h:h + 16] + gbuf[1, r, h:h + 16])
                    + (gbuf[2, r, h:h + 16] + gbuf[3, r, h:h + 16])) * 0.25

        pltpu.sync_copy(
            gbuf.at[4].at[pl.ds(0, nrows)],
            out.at[pl.ds(base, nrows), pl.ds(c * HALF_D, HALF_D)])

    @pl.loop(0, 24)
    def _fin(t):
        fin_chunk(t, 128)

    fin_chunk(24, 56)


_mesh = plsc.VectorSubcoreMesh(
    core_axis_name="c", subcore_axis_name="s", num_cores=NC, num_subcores=NS)

_tbl = jax.ShapeDtypeStruct((2 * N_TBL, HALF_D), jnp.float32)
_out_t = jax.ShapeDtypeStruct((N_TBL, 2 * HALF_D), jnp.float32)

_gcl = pl.kernel(
    _body,
    out_type=(_out_t, _tbl, _tbl),
    mesh=_mesh,
    compiler_params=pltpu.CompilerParams(
        use_tc_tiling_on_sc=False, needs_layout_passes=False),
    scratch_types=[
        pltpu.VMEM_SHARED((N_TBL, HALF_D), jnp.float32),
        pltpu.VMEM((NBUF, 3, 128), jnp.int32),
        pltpu.VMEM((NBUF, 128, HALF_D), jnp.float32),
        pltpu.SemaphoreType.DMA((NBUF,)),
        pltpu.SemaphoreType.DMA((NBUF,)),
        pltpu.SemaphoreType.DMA((NBUF,)),
        pltpu.SemaphoreType.DMA((4,)),
    ],
)


@jax.jit
def kernel(user_emb, item_emb, adj_rows, adj_cols, adj_vals):
    ego = jnp.concatenate([user_emb, item_emb], axis=0)
    zrows = jnp.zeros((N_TBL - N_NODES, HALF_D), jnp.float32)
    ego_h = jnp.concatenate(
        [ego[:, :HALF_D], zrows, ego[:, HALF_D:], zrows], axis=0)
    pad = E_PAD - E
    rows_p = jnp.concatenate(
        [adj_rows.astype(jnp.int32), jnp.zeros((pad,), jnp.int32)])
    cols_p = jnp.concatenate(
        [adj_cols.astype(jnp.int32), jnp.zeros((pad,), jnp.int32)])
    vals_p = jnp.concatenate([adj_vals, jnp.zeros((pad,), jnp.float32)])
    cols_r = cols_p.reshape(IDX_ROWS, 128)
    rows_r = rows_p.reshape(IDX_ROWS, 128)
    vals_r = lax.bitcast_convert_type(vals_p, jnp.int32).reshape(
        IDX_ROWS, 128)

    out, _, _ = _gcl(ego_h, cols_r, rows_r, vals_r)
    return out[: N_NODES // 2], out[N_NODES // 2: N_NODES]
